# baseline (speedup 1.0000x reference)
import functools

import jax
import jax.numpy as jnp
from jax import lax
from jax.experimental import pallas as pl
from jax.experimental.pallas import tpu as pltpu
from jax.experimental.pallas import tpu_sc as plsc

N = 10000
E = 160000
F_IN = 256
HID = 16
NCLS = 16

NC = 2
NS = 16
NW = NC * NS
CHUNK = 128
NCHUNK = 40
EPT = CHUNK * NCHUNK
E_PAD = EPT * NW
TOTCHUNK = E_PAD // CHUNK
NPAD = 10240
NPT = NPAD // NS
NROW = N * HID // 128
PROW = NPAD * HID // 128
EN_TOT = E + N

_SC_PARAMS = None


def _sc_params():
    global _SC_PARAMS
    if _SC_PARAMS is None:
        _SC_PARAMS = pltpu.CompilerParams(
            needs_layout_passes=False, use_tc_tiling_on_sc=False)
    return _SC_PARAMS


@functools.cache
def _get_mesh():
    return plsc.VectorSubcoreMesh(
        core_axis_name="c", subcore_axis_name="s",
        num_cores=NC, num_subcores=NS)


def _rsqrt16(d):
    i = plsc.bitcast(d, jnp.int32)
    i = jnp.int32(0x5F3759DF) - (i >> 1)
    y = plsc.bitcast(i, jnp.float32)
    for _ in range(3):
        y = y * (1.5 - 0.5 * d * y * y)
    return y


@functools.cache
def _make_conv_kernel(fused_deg):
    outs = [jax.ShapeDtypeStruct((NC, NPAD, HID), jnp.float32)]
    if fused_deg:
        outs += [
            jax.ShapeDtypeStruct((EN_TOT,), jnp.float32),
            jax.ShapeDtypeStruct((NPAD,), jnp.float32),
        ]
    scratch = [
        pltpu.VMEM((NCHUNK, CHUNK), jnp.int32),
        pltpu.VMEM((NCHUNK, CHUNK), jnp.int32),
        pltpu.VMEM((NCHUNK, CHUNK), jnp.float32),
        pltpu.VMEM((EPT, HID), jnp.float32),
        pltpu.VMEM((NPAD,), jnp.float32),
        pltpu.VMEM((EPT,), jnp.float32),
        pltpu.VMEM_SHARED((NPAD, HID), jnp.float32),
    ]
    if fused_deg:
        scratch += [
            pltpu.VMEM_SHARED((NPAD,), jnp.float32),
            pltpu.VMEM_SHARED((NPAD,), jnp.float32),
        ]
    scratch += [pltpu.SemaphoreType.DMA, pltpu.SemaphoreType.DMA]

    def body(h_hbm, *rest):
        if fused_deg:
            (row2_hbm, col2_hbm, ew2_hbm, zeros_hbm,
             pout_hbm, enew_hbm, dinv_hbm,
             row_v, col_v, ew_v, rows_v, dinv_v, enew_v,
             acc_sh, deg_sh, dinv_sh, gsem, ssem) = rest
        else:
            (dinv_hbm, row2_hbm, col2_hbm, ew2_hbm, zeros_hbm,
             pout_hbm,
             row_v, col_v, ew_v, rows_v, dinv_v, enew_v,
             acc_sh, gsem, ssem) = rest
        cidx = lax.axis_index("c")
        sid = lax.axis_index("s")
        wid = cidx * NS + sid
        cbase = wid * NCHUNK

        pltpu.sync_copy(row2_hbm.at[pl.ds(cbase, NCHUNK)], row_v)
        gdescs = [
            pltpu.async_copy(h_hbm.at[row_v.at[j]],
                             rows_v.at[pl.ds(j * CHUNK, CHUNK)], gsem)
            for j in range(NCHUNK)
        ]

        pltpu.sync_copy(zeros_hbm.at[pl.ds(sid * NPT, NPT)],
                        acc_sh.at[pl.ds(sid * NPT, NPT)])

        if fused_deg:
            def zbody(i, _):
                enew_v[pl.ds(i * 16, 16)] = jnp.zeros((16,), jnp.float32)
                return 0
            lax.fori_loop(0, NPT // 16, zbody, 0)
            pltpu.sync_copy(enew_v.at[pl.ds(0, NPT)],
                            deg_sh.at[pl.ds(sid * NPT, NPT)])
            plsc.subcore_barrier()

            for r in range(2):
                dbase = sid * (2 * NCHUNK) + r * NCHUNK
                pltpu.sync_copy(col2_hbm.at[pl.ds(dbase, NCHUNK)], col_v)
                pltpu.sync_copy(ew2_hbm.at[pl.ds(dbase, NCHUNK)], ew_v)
                ddescs = [
                    pltpu.async_copy(ew_v.at[j], deg_sh.at[col_v.at[j]],
                                     ssem, add=True)
                    for j in range(NCHUNK)
                ]
                for d in ddescs:
                    d.wait()
            plsc.subcore_barrier()

            pltpu.sync_copy(deg_sh.at[pl.ds(sid * NPT, NPT)],
                            enew_v.at[pl.ds(0, NPT)])

            def dbody(k, _):
                d16 = enew_v[pl.ds(k * 16, 16)] + 1.0
                y = _rsqrt16(d16)
                enew_v[pl.ds(NPT + k * 16, 16)] = y
                enew_v[pl.ds(2 * NPT + k * 16, 16)] = y * y
                return 0
            lax.fori_loop(0, NPT // 16, dbody, 0)
            pltpu.sync_copy(enew_v.at[pl.ds(NPT, NPT)],
                            dinv_sh.at[pl.ds(sid * NPT, NPT)])

            @pl.when(cidx == 0)
            def _():
                pltpu.sync_copy(enew_v.at[pl.ds(NPT, NPT)],
                                dinv_hbm.at[pl.ds(sid * NPT, NPT)])

                @pl.when(sid < NS - 1)
                def _():
                    pltpu.sync_copy(
                        enew_v.at[pl.ds(2 * NPT, NPT)],
                        enew_hbm.at[pl.ds(E + sid * NPT, NPT)])

                @pl.when(sid == NS - 1)
                def _():
                    pltpu.sync_copy(
                        enew_v.at[pl.ds(2 * NPT, N - (NS - 1) * NPT)],
                        enew_hbm.at[pl.ds(E + (NS - 1) * NPT,
                                          N - (NS - 1) * NPT)])

            plsc.subcore_barrier()
            pltpu.sync_copy(dinv_sh, dinv_v)
        else:
            plsc.subcore_barrier()
            pltpu.sync_copy(dinv_hbm, dinv_v)

        pltpu.sync_copy(col2_hbm.at[pl.ds(cbase, NCHUNK)], col_v)
        pltpu.sync_copy(ew2_hbm.at[pl.ds(cbase, NCHUNK)], ew_v)

        def gbody(i, _):
            j = i >> 3
            k = i & 7
            row16 = row_v[j, pl.ds(k * 16, 16)]
            ew16 = ew_v[j, pl.ds(k * 16, 16)]
            dr = plsc.load_gather(dinv_v, [row16])
            s16 = dr * ew16
            ew_v[j, pl.ds(k * 16, 16)] = s16
            if fused_deg:
                col16 = col_v[j, pl.ds(k * 16, 16)]
                dc = plsc.load_gather(dinv_v, [col16])
                enew_v[pl.ds(i * 16, 16)] = s16 * dc
            return 0
        lax.fori_loop(0, EPT // 16, gbody, 0)
        if fused_deg:
            @pl.when(wid < NW - 1)
            def _():
                pltpu.sync_copy(enew_v, enew_hbm.at[pl.ds(wid * EPT, EPT)])

            @pl.when(wid == NW - 1)
            def _():
                pltpu.sync_copy(
                    enew_v.at[pl.ds(0, E - (NW - 1) * EPT)],
                    enew_hbm.at[pl.ds((NW - 1) * EPT, E - (NW - 1) * EPT)])

        for d in gdescs:
            d.wait()

        def sbody(i, _):
            j = i >> 3
            k = i & 7
            s16 = ew_v[j, pl.ds(k * 16, 16)]
            for u in range(16):
                e = i * 16 + u
                rows_v[e] = rows_v[e] * s16[u]
            return 0
        lax.fori_loop(0, EPT // 16, sbody, 0)

        sdescs = [
            pltpu.async_copy(rows_v.at[pl.ds(j * CHUNK, CHUNK)],
                             acc_sh.at[col_v.at[j]], ssem, add=True)
            for j in range(NCHUNK)
        ]
        for d in sdescs:
            d.wait()
        plsc.subcore_barrier()

        pltpu.sync_copy(acc_sh.at[pl.ds(sid * NPT, NPT)],
                        pout_hbm.at[cidx, pl.ds(sid * NPT, NPT)])

    return pl.kernel(
        body, out_type=tuple(outs) if fused_deg else outs[0],
        mesh=_get_mesh(), scratch_types=tuple(scratch),
        compiler_params=_sc_params())


def _mm_body(x_ref, w_ref, o_ref):
    o_ref[...] = jnp.dot(x_ref[...], w_ref[...],
                         preferred_element_type=jnp.float32)


def _mid_body(p_ref, h_ref, dinv2d_ref, rep_ref, w2b_ref, b1t_ref, o_ref):
    drep = jnp.dot(dinv2d_ref[...], rep_ref[...],
                   preferred_element_type=jnp.float32,
                   precision=lax.Precision.HIGHEST)[:NROW, :]
    p = p_ref[0, :NROW, :] + p_ref[1, :NROW, :]
    out1 = drep * p + (drep * drep) * h_ref[...] + b1t_ref[...]
    g = jnp.maximum(out1, 0.0)
    o_ref[...] = jnp.dot(g, w2b_ref[...], preferred_element_type=jnp.float32)


def _fin_body(p_ref, h2_ref, dinv2d_ref, rep_ref, b2t_ref, o_ref):
    drep = jnp.dot(dinv2d_ref[...], rep_ref[...],
                   preferred_element_type=jnp.float32,
                   precision=lax.Precision.HIGHEST)[:NROW, :]
    z = (drep * (p_ref[0, :NROW, :] + p_ref[1, :NROW, :])
         + (drep * drep) * h2_ref[...] + b2t_ref[...])
    parts = []
    for u in range(8):
        zu = z[:, u * NCLS:(u + 1) * NCLS]
        m = jnp.max(zu, axis=1, keepdims=True)
        ez = jnp.exp(zu - m)
        parts.append(zu - m - jnp.log(jnp.sum(ez, axis=1, keepdims=True)))
    o_ref[...] = jnp.concatenate(parts, axis=1)


def kernel(x, edge_index, edge_weight, W1, b1, W2, b2):
    row = edge_index[0]
    col = edge_index[1]
    pad = E_PAD - E
    rowp = jnp.concatenate([row, jnp.zeros((pad,), jnp.int32)])
    colp = jnp.concatenate([col, jnp.zeros((pad,), jnp.int32)])
    ewp = jnp.concatenate([edge_weight, jnp.zeros((pad,), jnp.float32)])

    row2d = rowp.reshape(TOTCHUNK, CHUNK)
    col2d = colp.reshape(TOTCHUNK, CHUNK)
    ew2d = ewp.reshape(TOTCHUNK, CHUNK)
    zeros2d = jnp.zeros((NPAD, HID), jnp.float32)

    W2big = jnp.kron(jnp.eye(8, dtype=jnp.float32), W2)
    Rrep = jnp.kron(jnp.eye(8, dtype=jnp.float32),
                    jnp.ones((1, HID), jnp.float32))
    b1t = jnp.tile(b1, 8).reshape(1, 128)
    b2t = jnp.tile(b2, 8).reshape(1, 128)

    h1 = pl.pallas_call(
        _mm_body,
        out_shape=jax.ShapeDtypeStruct((N, HID), jnp.float32),
    )(x, W1)
    h1p = h1.reshape(NROW, 128)

    p1, e_new, dinv_p = _make_conv_kernel(True)(
        h1, row2d, col2d, ew2d, zeros2d)

    p1p = p1.reshape(NC, PROW, 128)
    dinv2d = dinv_p.reshape(NPAD // 8, 8)

    h2p = pl.pallas_call(
        _mid_body,
        out_shape=jax.ShapeDtypeStruct((NROW, 128), jnp.float32),
    )(p1p, h1p, dinv2d, Rrep, W2big, b1t)

    h2src = h2p.reshape(N, HID)
    p2 = _make_conv_kernel(False)(h2src, dinv_p, row2d, col2d, ew2d, zeros2d)
    p2p = p2.reshape(NC, PROW, 128)

    outp = pl.pallas_call(
        _fin_body,
        out_shape=jax.ShapeDtypeStruct((NROW, 128), jnp.float32),
    )(p2p, h2p, dinv2d, Rrep, b2t)

    return outp.reshape(N, NCLS), e_new


# speedup vs baseline: 1.0338x; 1.0338x over previous
import functools

import jax
import jax.numpy as jnp
from jax import lax
from jax.experimental import pallas as pl
from jax.experimental.pallas import tpu as pltpu
from jax.experimental.pallas import tpu_sc as plsc

N = 10000
E = 160000
F_IN = 256
HID = 16
NCLS = 16

NC = 2
NS = 16
NW = NC * NS
CHUNK = 128
NCHUNK = 40
EPT = CHUNK * NCHUNK
E_PAD = EPT * NW
TOTCHUNK = E_PAD // CHUNK
NPAD = 10240
NPT = NPAD // NS
NROW = N * HID // 128
PROW = NPAD * HID // 128
EN_TOT = E + N

_SC_PARAMS = None


def _sc_params():
    global _SC_PARAMS
    if _SC_PARAMS is None:
        _SC_PARAMS = pltpu.CompilerParams(
            needs_layout_passes=False, use_tc_tiling_on_sc=False)
    return _SC_PARAMS


@functools.cache
def _get_mesh():
    return plsc.VectorSubcoreMesh(
        core_axis_name="c", subcore_axis_name="s",
        num_cores=NC, num_subcores=NS)


def _rsqrt16(d):
    i = plsc.bitcast(d, jnp.int32)
    i = jnp.int32(0x5F3759DF) - (i >> 1)
    y = plsc.bitcast(i, jnp.float32)
    for _ in range(3):
        y = y * (1.5 - 0.5 * d * y * y)
    return y


@functools.cache
def _make_conv_kernel(fused_deg):
    outs = [jax.ShapeDtypeStruct((NC, NPAD, HID), jnp.float32)]
    if fused_deg:
        outs += [
            jax.ShapeDtypeStruct((EN_TOT,), jnp.float32),
            jax.ShapeDtypeStruct((NPAD,), jnp.float32),
            jax.ShapeDtypeStruct((NPAD, HID), jnp.float32),
            jax.ShapeDtypeStruct((NPAD, HID), jnp.float32),
        ]
    scratch = [
        pltpu.VMEM((NCHUNK, CHUNK), jnp.int32),
        pltpu.VMEM((NCHUNK, CHUNK), jnp.int32),
        pltpu.VMEM((NCHUNK, CHUNK), jnp.float32),
        pltpu.VMEM((EPT, HID), jnp.float32),
        pltpu.VMEM((NPAD,), jnp.float32),
        pltpu.VMEM((EPT,), jnp.float32),
        pltpu.VMEM_SHARED((NPAD, HID), jnp.float32),
    ]
    if fused_deg:
        scratch += [
            pltpu.VMEM((NPT // 2, HID), jnp.float32),
            pltpu.VMEM_SHARED((NPAD,), jnp.float32),
            pltpu.VMEM_SHARED((NPAD,), jnp.float32),
        ]
    scratch += [pltpu.SemaphoreType.DMA, pltpu.SemaphoreType.DMA]

    def body(h_hbm, *rest):
        if fused_deg:
            (row2_hbm, col2_hbm, ew2_hbm, zeros_hbm,
             pout_hbm, enew_hbm, dinv_hbm, drep_hbm, drep2_hbm,
             row_v, col_v, ew_v, rows_v, dinv_v, enew_v,
             acc_sh, rep_v, deg_sh, dinv_sh, gsem, ssem) = rest
        else:
            (dinv_hbm, row2_hbm, col2_hbm, ew2_hbm, zeros_hbm,
             pout_hbm,
             row_v, col_v, ew_v, rows_v, dinv_v, enew_v,
             acc_sh, gsem, ssem) = rest
        cidx = lax.axis_index("c")
        sid = lax.axis_index("s")
        wid = cidx * NS + sid
        cbase = wid * NCHUNK

        pltpu.sync_copy(row2_hbm.at[pl.ds(cbase, NCHUNK)], row_v)
        gdescs = [
            pltpu.async_copy(h_hbm.at[row_v.at[j]],
                             rows_v.at[pl.ds(j * CHUNK, CHUNK)], gsem)
            for j in range(NCHUNK)
        ]

        pltpu.sync_copy(zeros_hbm.at[pl.ds(sid * NPT, NPT)],
                        acc_sh.at[pl.ds(sid * NPT, NPT)])

        if fused_deg:
            def zbody(i, _):
                enew_v[pl.ds(i * 16, 16)] = jnp.zeros((16,), jnp.float32)
                return 0
            lax.fori_loop(0, NPT // 16, zbody, 0)
            pltpu.sync_copy(enew_v.at[pl.ds(0, NPT)],
                            deg_sh.at[pl.ds(sid * NPT, NPT)])
            plsc.subcore_barrier()

            for r in range(2):
                dbase = sid * (2 * NCHUNK) + r * NCHUNK
                pltpu.sync_copy(col2_hbm.at[pl.ds(dbase, NCHUNK)], col_v)
                pltpu.sync_copy(ew2_hbm.at[pl.ds(dbase, NCHUNK)], ew_v)
                ddescs = [
                    pltpu.async_copy(ew_v.at[j], deg_sh.at[col_v.at[j]],
                                     ssem, add=True)
                    for j in range(NCHUNK)
                ]
                for d in ddescs:
                    d.wait()
            plsc.subcore_barrier()

            pltpu.sync_copy(deg_sh.at[pl.ds(sid * NPT, NPT)],
                            enew_v.at[pl.ds(0, NPT)])

            def dbody(k, _):
                d16 = enew_v[pl.ds(k * 16, 16)] + 1.0
                y = _rsqrt16(d16)
                enew_v[pl.ds(NPT + k * 16, 16)] = y
                enew_v[pl.ds(2 * NPT + k * 16, 16)] = y * y
                return 0
            lax.fori_loop(0, NPT // 16, dbody, 0)
            pltpu.sync_copy(enew_v.at[pl.ds(NPT, NPT)],
                            dinv_sh.at[pl.ds(sid * NPT, NPT)])

            @pl.when(cidx == 0)
            def _():
                pltpu.sync_copy(enew_v.at[pl.ds(NPT, NPT)],
                                dinv_hbm.at[pl.ds(sid * NPT, NPT)])

                @pl.when(sid < NS - 1)
                def _():
                    pltpu.sync_copy(
                        enew_v.at[pl.ds(2 * NPT, NPT)],
                        enew_hbm.at[pl.ds(E + sid * NPT, NPT)])

                @pl.when(sid == NS - 1)
                def _():
                    pltpu.sync_copy(
                        enew_v.at[pl.ds(2 * NPT, N - (NS - 1) * NPT)],
                        enew_hbm.at[pl.ds(E + (NS - 1) * NPT,
                                          N - (NS - 1) * NPT)])

            zero16 = jnp.zeros((16,), jnp.float32)

            def rep(src_off, dst_hbm):
                for b in range(2):
                    def rbody(k, _):
                        y = enew_v[pl.ds(src_off + b * (NPT // 2)
                                         + k * 16, 16)]
                        for u in range(16):
                            rep_v[k * 16 + u, :] = zero16 + y[u]
                        return 0
                    lax.fori_loop(0, NPT // 32, rbody, 0)
                    pltpu.sync_copy(
                        rep_v,
                        dst_hbm.at[pl.ds(sid * NPT + b * (NPT // 2),
                                         NPT // 2)])

            @pl.when(cidx == 1)
            def _():
                rep(NPT, drep_hbm)

            @pl.when(cidx == 0)
            def _():
                rep(2 * NPT, drep2_hbm)

            plsc.subcore_barrier()
            pltpu.sync_copy(dinv_sh, dinv_v)
        else:
            plsc.subcore_barrier()
            pltpu.sync_copy(dinv_hbm, dinv_v)

        pltpu.sync_copy(col2_hbm.at[pl.ds(cbase, NCHUNK)], col_v)
        pltpu.sync_copy(ew2_hbm.at[pl.ds(cbase, NCHUNK)], ew_v)

        def gbody(i, _):
            j = i >> 3
            k = i & 7
            row16 = row_v[j, pl.ds(k * 16, 16)]
            ew16 = ew_v[j, pl.ds(k * 16, 16)]
            dr = plsc.load_gather(dinv_v, [row16])
            s16 = dr * ew16
            ew_v[j, pl.ds(k * 16, 16)] = s16
            if fused_deg:
                col16 = col_v[j, pl.ds(k * 16, 16)]
                dc = plsc.load_gather(dinv_v, [col16])
                enew_v[pl.ds(i * 16, 16)] = s16 * dc
            return 0
        lax.fori_loop(0, EPT // 16, gbody, 0)
        if fused_deg:
            @pl.when(wid < NW - 1)
            def _():
                pltpu.sync_copy(enew_v, enew_hbm.at[pl.ds(wid * EPT, EPT)])

            @pl.when(wid == NW - 1)
            def _():
                pltpu.sync_copy(
                    enew_v.at[pl.ds(0, E - (NW - 1) * EPT)],
                    enew_hbm.at[pl.ds((NW - 1) * EPT, E - (NW - 1) * EPT)])

        for d in gdescs:
            d.wait()

        def sbody(i, _):
            j = i >> 3
            k = i & 7
            s16 = ew_v[j, pl.ds(k * 16, 16)]
            for u in range(16):
                e = i * 16 + u
                rows_v[e] = rows_v[e] * s16[u]
            return 0
        lax.fori_loop(0, EPT // 16, sbody, 0)

        sdescs = [
            pltpu.async_copy(rows_v.at[pl.ds(j * CHUNK, CHUNK)],
                             acc_sh.at[col_v.at[j]], ssem, add=True)
            for j in range(NCHUNK)
        ]
        for d in sdescs:
            d.wait()
        plsc.subcore_barrier()

        pltpu.sync_copy(acc_sh.at[pl.ds(sid * NPT, NPT)],
                        pout_hbm.at[cidx, pl.ds(sid * NPT, NPT)])

    return pl.kernel(
        body, out_type=tuple(outs) if fused_deg else outs[0],
        mesh=_get_mesh(), scratch_types=tuple(scratch),
        compiler_params=_sc_params())


def _mm_body(x_ref, w_ref, o_ref):
    o_ref[...] = jnp.dot(x_ref[...], w_ref[...],
                         preferred_element_type=jnp.float32)


def _mid_body(p_ref, h_ref, drep_ref, drep2_ref, w2b_ref, b1t_ref, o_ref):
    p = p_ref[0, :NROW, :] + p_ref[1, :NROW, :]
    out1 = (drep_ref[:NROW, :] * p + drep2_ref[:NROW, :] * h_ref[...]
            + b1t_ref[...])
    g = jnp.maximum(out1, 0.0)
    o_ref[...] = jnp.dot(g, w2b_ref[...], preferred_element_type=jnp.float32)


def _fin_body(p_ref, h2_ref, drep_ref, drep2_ref, b2t_ref, o_ref):
    z = (drep_ref[:NROW, :] * (p_ref[0, :NROW, :] + p_ref[1, :NROW, :])
         + drep2_ref[:NROW, :] * h2_ref[...] + b2t_ref[...])
    parts = []
    for u in range(8):
        zu = z[:, u * NCLS:(u + 1) * NCLS]
        m = jnp.max(zu, axis=1, keepdims=True)
        ez = jnp.exp(zu - m)
        parts.append(zu - m - jnp.log(jnp.sum(ez, axis=1, keepdims=True)))
    o_ref[...] = jnp.concatenate(parts, axis=1)


def kernel(x, edge_index, edge_weight, W1, b1, W2, b2):
    row = edge_index[0]
    col = edge_index[1]
    pad = E_PAD - E
    rowp = jnp.concatenate([row, jnp.zeros((pad,), jnp.int32)])
    colp = jnp.concatenate([col, jnp.zeros((pad,), jnp.int32)])
    ewp = jnp.concatenate([edge_weight, jnp.zeros((pad,), jnp.float32)])

    row2d = rowp.reshape(TOTCHUNK, CHUNK)
    col2d = colp.reshape(TOTCHUNK, CHUNK)
    ew2d = ewp.reshape(TOTCHUNK, CHUNK)
    zeros2d = jnp.zeros((NPAD, HID), jnp.float32)

    W2big = jnp.kron(jnp.eye(8, dtype=jnp.float32), W2)
    b1t = jnp.tile(b1, 8).reshape(1, 128)
    b2t = jnp.tile(b2, 8).reshape(1, 128)

    h1 = pl.pallas_call(
        _mm_body,
        out_shape=jax.ShapeDtypeStruct((N, HID), jnp.float32),
    )(x, W1)
    h1p = h1.reshape(NROW, 128)

    p1, e_new, dinv_p, drep, drep2 = _make_conv_kernel(True)(
        h1, row2d, col2d, ew2d, zeros2d)

    p1p = p1.reshape(NC, PROW, 128)
    drepP = drep.reshape(PROW, 128)
    drep2P = drep2.reshape(PROW, 128)

    h2p = pl.pallas_call(
        _mid_body,
        out_shape=jax.ShapeDtypeStruct((NROW, 128), jnp.float32),
    )(p1p, h1p, drepP, drep2P, W2big, b1t)

    h2src = h2p.reshape(N, HID)
    p2 = _make_conv_kernel(False)(h2src, dinv_p, row2d, col2d, ew2d, zeros2d)
    p2p = p2.reshape(NC, PROW, 128)

    outp = pl.pallas_call(
        _fin_body,
        out_shape=jax.ShapeDtypeStruct((NROW, 128), jnp.float32),
    )(p2p, h2p, drepP, drep2P, b2t)

    return outp.reshape(N, NCLS), e_new
